# baseline (speedup 1.0000x reference)
import jax
import jax.numpy as jnp
from jax import lax
from jax.experimental import pallas as pl
from jax.experimental.pallas import tpu as pltpu, tpu_sc as plsc

D = 256
N = 8192
NC = 2
NS = 16
NW = NC * NS
RPW = N // NW
L = 16
NQ = 4
CR = RPW // NQ

_MESH = plsc.VectorSubcoreMesh(
    core_axis_name="c", subcore_axis_name="s", num_cores=NC, num_subcores=NS)
_PARAMS = pltpu.CompilerParams(needs_layout_passes=False)


def _stage1(cb_hbm, z_hbm, out_d, out_i, cb_v, z_v, res_v, resi_v, sems):
    c = lax.axis_index("c")
    s = lax.axis_index("s")
    wid = c * NS + s
    base_row = wid * RPW

    copies = [
        pltpu.async_copy(
            cb_hbm.at[pl.ds(base_row + q * CR, CR)], cb_v.at[q], sems.at[q])
        for q in range(NQ)
    ]
    pltpu.sync_copy(z_hbm, z_v)
    z_vecs = [z_v[0, pl.ds(L * d, L)] for d in range(D // L)]

    iota = lax.broadcasted_iota(jnp.int32, (L,), 0)

    def tree_reduce(vecs):
        for k in range(4):
            bit = 1 << k
            mask = (iota & bit) != 0
            perm = iota ^ bit
            nxt = []
            for i in range(0, len(vecs), 2):
                a, b = vecs[i], vecs[i + 1]
                sel = jnp.where(mask, b, a)
                cross = jnp.where(
                    mask,
                    a.at[perm].get(mode="promise_in_bounds"),
                    b.at[perm].get(mode="promise_in_bounds"))
                nxt.append(sel + cross)
            vecs = nxt
        return vecs[0]

    best16 = jnp.full((L,), jnp.inf, jnp.float32)
    bidx16 = jnp.zeros((L,), jnp.int32)
    for q in range(NQ):
        copies[q].wait()

        def group_body(g, carry, q=q):
            best16, bidx16 = carry
            accs = []
            for r in range(L):
                acc = jnp.zeros((L,), jnp.float32)
                for d in range(D // L):
                    t = cb_v[q, g * L + r, pl.ds(L * d, L)] - z_vecs[d]
                    acc = acc + t * t
                accs.append(acc)
            w = tree_reduce(accs)
            gidx = (base_row + q * CR + g * L) + iota
            m = w < best16
            best16 = jnp.where(m, w, best16)
            bidx16 = jnp.where(m, gidx, bidx16)
            return best16, bidx16

        best16, bidx16 = lax.fori_loop(0, CR // L, group_body,
                                       (best16, bidx16))

    mn = jnp.min(best16)
    gi = jnp.min(jnp.where(best16 == mn, bidx16, jnp.int32(N)))
    res_v[...] = jnp.broadcast_to(mn, (L,))
    resi_v[...] = jnp.broadcast_to(gi, (L,))
    pltpu.sync_copy(res_v, out_d.at[pl.ds(wid * L, L)])
    pltpu.sync_copy(resi_v, out_i.at[pl.ds(wid * L, L)])


def _merge(d_ref, i_ref, cb_any, row_ref, idx_ref, dist_ref, rowbuf, sem):
    d = d_ref[...]
    i = i_ref[...]
    dmin = jnp.min(d)
    idx = jnp.min(jnp.where(d == dmin, i, jnp.int32(N)))
    cp = pltpu.make_async_copy(cb_any.at[pl.ds(idx, 1)], rowbuf, sem)
    cp.start()
    idx_ref[0, 0] = idx
    dist_ref[0, 0] = jnp.sqrt(dmin)
    cp.wait()
    row_ref[...] = rowbuf[...]


@jax.jit
def kernel(z, codebook):
    out_d, out_i = pl.kernel(
        _stage1,
        out_type=(
            jax.ShapeDtypeStruct((NW * L,), jnp.float32),
            jax.ShapeDtypeStruct((NW * L,), jnp.int32),
        ),
        mesh=_MESH,
        compiler_params=_PARAMS,
        scratch_types=[
            pltpu.VMEM((NQ, CR, D), jnp.float32),
            pltpu.VMEM((1, D), jnp.float32),
            pltpu.VMEM((L,), jnp.float32),
            pltpu.VMEM((L,), jnp.int32),
            pltpu.SemaphoreType.DMA((NQ,)),
        ],
    )(codebook, z)

    row, idx, dist = pl.pallas_call(
        _merge,
        out_shape=(
            jax.ShapeDtypeStruct((1, D), jnp.float32),
            jax.ShapeDtypeStruct((1, 1), jnp.int32),
            jax.ShapeDtypeStruct((1, 1), jnp.float32),
        ),
        in_specs=[
            pl.BlockSpec(memory_space=pltpu.VMEM),
            pl.BlockSpec(memory_space=pltpu.VMEM),
            pl.BlockSpec(memory_space=pl.ANY),
        ],
        out_specs=(
            pl.BlockSpec(memory_space=pltpu.VMEM),
            pl.BlockSpec(memory_space=pltpu.SMEM),
            pl.BlockSpec(memory_space=pltpu.SMEM),
        ),
        scratch_shapes=[
            pltpu.VMEM((1, D), jnp.float32),
            pltpu.SemaphoreType.DMA,
        ],
    )(out_d, out_i, codebook)

    return row[0], idx[0, 0], dist[0, 0]


# speedup vs baseline: 1.0032x; 1.0032x over previous
import jax
import jax.numpy as jnp
from jax import lax
from jax.experimental import pallas as pl
from jax.experimental.pallas import tpu as pltpu, tpu_sc as plsc

D = 256
N = 8192
NC = 2
NS = 16
NW = NC * NS
RPW = N // NW
L = 16
NQ = 4
CR = RPW // NQ

_MESH = plsc.VectorSubcoreMesh(
    core_axis_name="c", subcore_axis_name="s", num_cores=NC, num_subcores=NS)
_PARAMS = pltpu.CompilerParams(needs_layout_passes=False)


def _stage1(cb_hbm, z_hbm, out_d, out_i, cb_v, z_v, res_v, resi_v, sems):
    c = lax.axis_index("c")
    s = lax.axis_index("s")
    wid = c * NS + s
    base_row = wid * RPW

    copies = [
        pltpu.async_copy(
            cb_hbm.at[pl.ds(base_row + q * CR, CR)], cb_v.at[q], sems.at[q])
        for q in range(NQ)
    ]
    pltpu.sync_copy(z_hbm, z_v)
    z_vecs = [z_v[0, pl.ds(L * d, L)] for d in range(D // L)]

    iota = lax.broadcasted_iota(jnp.int32, (L,), 0)

    def tree_reduce(vecs):
        for k in range(4):
            bit = 1 << k
            mask = (iota & bit) != 0
            perm = iota ^ bit
            nxt = []
            for i in range(0, len(vecs), 2):
                a, b = vecs[i], vecs[i + 1]
                sel = jnp.where(mask, b, a)
                cross = jnp.where(
                    mask,
                    b.at[perm].get(mode="promise_in_bounds"),
                    a.at[perm].get(mode="promise_in_bounds"))
                nxt.append(sel + cross)
            vecs = nxt
        return vecs[0]

    best16 = jnp.full((L,), jnp.inf, jnp.float32)
    bidx16 = jnp.zeros((L,), jnp.int32)
    for q in range(NQ):
        copies[q].wait()

        def group_body(g, carry, q=q):
            best16, bidx16 = carry
            accs = []
            for r in range(L):
                acc = jnp.zeros((L,), jnp.float32)
                for d in range(D // L):
                    t = cb_v[q, g * L + r, pl.ds(L * d, L)] - z_vecs[d]
                    acc = acc + t * t
                accs.append(acc)
            w = tree_reduce(accs)
            gidx = (base_row + q * CR + g * L) + iota
            m = w < best16
            best16 = jnp.where(m, w, best16)
            bidx16 = jnp.where(m, gidx, bidx16)
            return best16, bidx16

        best16, bidx16 = lax.fori_loop(0, CR // L, group_body,
                                       (best16, bidx16))

    mn = jnp.min(best16)
    gi = jnp.min(jnp.where(best16 == mn, bidx16, jnp.int32(N)))
    res_v[...] = jnp.broadcast_to(mn, (L,))
    resi_v[...] = jnp.broadcast_to(gi, (L,))
    pltpu.sync_copy(res_v, out_d.at[pl.ds(wid * L, L)])
    pltpu.sync_copy(resi_v, out_i.at[pl.ds(wid * L, L)])


def _merge(d_ref, i_ref, cb_any, row_ref, idx_ref, dist_ref, rowbuf, sem):
    d = d_ref[...]
    i = i_ref[...]
    dmin = jnp.min(d)
    idx = jnp.min(jnp.where(d == dmin, i, jnp.int32(N)))
    cp = pltpu.make_async_copy(cb_any.at[pl.ds(idx, 1)], rowbuf, sem)
    cp.start()
    idx_ref[0, 0] = idx
    dist_ref[0, 0] = jnp.sqrt(dmin)
    cp.wait()
    row_ref[...] = rowbuf[...]


@jax.jit
def kernel(z, codebook):
    out_d, out_i = pl.kernel(
        _stage1,
        out_type=(
            jax.ShapeDtypeStruct((NW * L,), jnp.float32),
            jax.ShapeDtypeStruct((NW * L,), jnp.int32),
        ),
        mesh=_MESH,
        compiler_params=_PARAMS,
        scratch_types=[
            pltpu.VMEM((NQ, CR, D), jnp.float32),
            pltpu.VMEM((1, D), jnp.float32),
            pltpu.VMEM((L,), jnp.float32),
            pltpu.VMEM((L,), jnp.int32),
            pltpu.SemaphoreType.DMA((NQ,)),
        ],
    )(codebook, z)

    row, idx, dist = pl.pallas_call(
        _merge,
        out_shape=(
            jax.ShapeDtypeStruct((1, D), jnp.float32),
            jax.ShapeDtypeStruct((1, 1), jnp.int32),
            jax.ShapeDtypeStruct((1, 1), jnp.float32),
        ),
        in_specs=[
            pl.BlockSpec(memory_space=pltpu.VMEM),
            pl.BlockSpec(memory_space=pltpu.VMEM),
            pl.BlockSpec(memory_space=pl.ANY),
        ],
        out_specs=(
            pl.BlockSpec(memory_space=pltpu.VMEM),
            pl.BlockSpec(memory_space=pltpu.SMEM),
            pl.BlockSpec(memory_space=pltpu.SMEM),
        ),
        scratch_shapes=[
            pltpu.VMEM((1, D), jnp.float32),
            pltpu.SemaphoreType.DMA,
        ],
    )(out_d, out_i, codebook)

    return row[0], idx[0, 0], dist[0, 0]
